# Initial kernel scaffold; baseline (speedup 1.0000x reference)
import jax
import jax.numpy as jnp
from jax import lax
from jax.experimental import pallas as pl
from jax.experimental.pallas import tpu as pltpu
from jax.experimental.pallas import tpu_sc as plsc

WORD_LEN = 20
CHAR_VOCAB = 1000
D = 64
N_NODES = 16384
L = 16
NC, NS = 2, 16
NW = NC * NS
NPT = N_NODES // NW
IDX_CHUNK = 128
N_IDX_CHUNKS = NPT // IDX_CHUNK
NG = NPT // L


def _sc_body(tokens_hbm, ids_hbm, table_hbm, out_hbm,
             ids_v, tok_v, table_v, out_v, sem):
    wid = lax.axis_index("s") * NC + lax.axis_index("c")

    pltpu.sync_copy(ids_hbm.at[wid], ids_v)
    pltpu.sync_copy(table_hbm, table_v)
    for j in range(N_IDX_CHUNKS):
        pltpu.async_copy(tokens_hbm.at[ids_v.at[j]],
                         tok_v.at[pl.ds(j * IDX_CHUNK, IDX_CHUNK)],
                         sem).wait()
    zeros = jnp.zeros((L,), jnp.float32)
    for j in range(D // L):
        table_v[pl.ds(j * L, L)] = zeros

    lane = lax.iota(jnp.int32, (L,))

    def group_body(g, carry):
        node = g * L + lane
        toks = [
            plsc.load_gather(tok_v, [node, jnp.full((L,), c, jnp.int32)])
            for c in range(WORD_LEN)
        ]
        one = jnp.ones((L,), jnp.float32)
        zero = jnp.zeros((L,), jnp.float32)
        cnt = zero
        for c in range(WORD_LEN):
            cnt = cnt + jnp.where(toks[c] != 0, one, zero)
        inv = one / jnp.maximum(cnt, one)
        toks64 = [t * D for t in toks]
        node64 = node * D
        for d in range(D):
            acc = plsc.load_gather(table_v, [toks64[0] + d])
            for c in range(1, WORD_LEN):
                acc = acc + plsc.load_gather(table_v, [toks64[c] + d])
            plsc.store_scatter(out_v, [node64 + d], acc * inv)
        return carry

    lax.fori_loop(0, NG, group_body, 0)

    pltpu.sync_copy(out_v, out_hbm.at[pl.ds(wid * NPT * D, NPT * D)])


@jax.jit
def _run(tokens, ids_slabs, table_flat):
    mesh = plsc.VectorSubcoreMesh(
        core_axis_name="c", subcore_axis_name="s",
        num_cores=NC, num_subcores=NS)
    f = pl.kernel(
        _sc_body,
        out_type=jax.ShapeDtypeStruct((N_NODES * D,), jnp.float32),
        mesh=mesh,
        scratch_types=[
            pltpu.VMEM((N_IDX_CHUNKS, IDX_CHUNK), jnp.int32),
            pltpu.VMEM((NPT, WORD_LEN), jnp.int32),
            pltpu.VMEM((CHAR_VOCAB * D,), jnp.float32),
            pltpu.VMEM((NPT * D,), jnp.float32),
            pltpu.SemaphoreType.DMA,
        ],
    )
    return f(tokens, ids_slabs, table_flat).reshape(N_NODES, D)


def kernel(local_char_embedding_tokens, lookup_ids, char_table, special_vectors):
    del special_vectors
    ids_slabs = lookup_ids.astype(jnp.int32).reshape(NW, N_IDX_CHUNKS, IDX_CHUNK)
    tokens = local_char_embedding_tokens.astype(jnp.int32)
    table_flat = char_table.reshape(CHAR_VOCAB * D)
    return _run(tokens, ids_slabs, table_flat)


# speedup vs baseline: 11.7635x; 11.7635x over previous
import jax
import jax.numpy as jnp
from jax import lax
from jax.experimental import pallas as pl
from jax.experimental.pallas import tpu as pltpu
from jax.experimental.pallas import tpu_sc as plsc

WORD_LEN = 20
CHAR_VOCAB = 1000
D = 64
N_NODES = 16384
L = 16
NC, NS = 2, 16
NW = NC * NS
NPT = N_NODES // NW
NG = NPT // L
IDX_CHUNK = 128
N_ELEM = NPT * WORD_LEN
N_CHUNKS = N_ELEM // IDX_CHUNK
DMA_BATCH = 8
N_BATCHES = N_CHUNKS // DMA_BATCH


def _sc_body(tokens_hbm, idxe_hbm, table_hbm, out_hbm,
             idxe_v, tok_v, table_v, out_v, sem):
    wid = lax.axis_index("s") * NC + lax.axis_index("c")

    pltpu.sync_copy(idxe_hbm.at[wid], idxe_v)
    pltpu.sync_copy(table_hbm, table_v)

    def dma_body(o, carry):
        copies = [
            pltpu.async_copy(
                tokens_hbm.at[idxe_v.at[o * DMA_BATCH + b]],
                tok_v.at[pl.ds((o * DMA_BATCH + b) * IDX_CHUNK, IDX_CHUNK)],
                sem)
            for b in range(DMA_BATCH)
        ]
        for cp in copies:
            cp.wait()
        return carry

    lax.fori_loop(0, N_BATCHES, dma_body, 0)

    zeros = jnp.zeros((L,), jnp.float32)
    for j in range(D // L):
        table_v[pl.ds(j * L, L)] = zeros

    lane = lax.iota(jnp.int32, L)
    one = jnp.ones((L,), jnp.float32)
    zero = jnp.zeros((L,), jnp.float32)

    def group_body(g, carry):
        node = g * L + lane
        base = g * L
        toks = [tok_v[pl.ds(c * NPT + base, L)] for c in range(WORD_LEN)]
        cnt = zero
        for c in range(WORD_LEN):
            cnt = cnt + jnp.where(toks[c] != 0, one, zero)
        inv = one / jnp.maximum(cnt, one)
        toks64 = [t * D for t in toks]
        node64 = node * D
        for d in range(D):
            acc = plsc.load_gather(table_v, [toks64[0] + d])
            for c in range(1, WORD_LEN):
                acc = acc + plsc.load_gather(table_v, [toks64[c] + d])
            plsc.store_scatter(out_v, [node64 + d], acc * inv)
        return carry

    lax.fori_loop(0, NG, group_body, 0)

    pltpu.sync_copy(out_v, out_hbm.at[pl.ds(wid * NPT * D, NPT * D)])


@jax.jit
def _run(tokens_flat, idxe, table_flat):
    mesh = plsc.VectorSubcoreMesh(
        core_axis_name="c", subcore_axis_name="s",
        num_cores=NC, num_subcores=NS)
    f = pl.kernel(
        _sc_body,
        out_type=jax.ShapeDtypeStruct((N_NODES * D,), jnp.float32),
        mesh=mesh,
        compiler_params=pltpu.CompilerParams(needs_layout_passes=False),
        scratch_types=[
            pltpu.VMEM((N_CHUNKS, IDX_CHUNK), jnp.int32),
            pltpu.VMEM((N_ELEM,), jnp.int32),
            pltpu.VMEM((CHAR_VOCAB * D,), jnp.float32),
            pltpu.VMEM((NPT * D,), jnp.float32),
            pltpu.SemaphoreType.DMA,
        ],
    )
    return f(tokens_flat, idxe, table_flat).reshape(N_NODES, D)


def kernel(local_char_embedding_tokens, lookup_ids, char_table, special_vectors):
    del special_vectors
    tokens_flat = local_char_embedding_tokens.astype(jnp.int32).reshape(-1)
    ids = lookup_ids.astype(jnp.int32).reshape(NW, 1, NPT)
    idxe = (ids * WORD_LEN
            + jnp.arange(WORD_LEN, dtype=jnp.int32).reshape(1, WORD_LEN, 1))
    idxe = idxe.reshape(NW, N_CHUNKS, IDX_CHUNK)
    table_flat = char_table.reshape(CHAR_VOCAB * D)
    return _run(tokens_flat, idxe, table_flat)


# speedup vs baseline: 22.9547x; 1.9513x over previous
import jax
import jax.numpy as jnp
from jax import lax
from jax.experimental import pallas as pl
from jax.experimental.pallas import tpu as pltpu
from jax.experimental.pallas import tpu_sc as plsc

WORD_LEN = 20
CHAR_VOCAB = 1000
D = 64
N_NODES = 16384
L = 16
NC, NS = 2, 16
NW = NC * NS
NPT = N_NODES // NW
NG = NPT // L
IDX_CHUNK = 128
N_ELEM = NPT * WORD_LEN
N_CHUNKS = N_ELEM // IDX_CHUNK
DMA_BATCH = 8
N_BATCHES = N_CHUNKS // DMA_BATCH
TSTRIDE = D + 1


def _sc_body(tokens_hbm, idxe_hbm, table_hbm, out_hbm,
             idxe_v, tok_v, table_v, out_v, sem):
    wid = lax.axis_index("s") * NC + lax.axis_index("c")

    pltpu.sync_copy(idxe_hbm.at[wid], idxe_v)
    pltpu.sync_copy(table_hbm, table_v)

    def dma_body(o, carry):
        copies = [
            pltpu.async_copy(
                tokens_hbm.at[idxe_v.at[o * DMA_BATCH + b]],
                tok_v.at[pl.ds((o * DMA_BATCH + b) * IDX_CHUNK, IDX_CHUNK)],
                sem)
            for b in range(DMA_BATCH)
        ]
        for cp in copies:
            cp.wait()
        return carry

    lax.fori_loop(0, N_BATCHES, dma_body, 0)

    zeros = jnp.zeros((L,), jnp.float32)
    for j in range(D // L):
        table_v[pl.ds(j * L, L)] = zeros

    one = jnp.ones((L,), jnp.float32)
    zero = jnp.zeros((L,), jnp.float32)

    def group_body(g, carry):
        base = g * L
        toks = [tok_v[pl.ds(c * NPT + base, L)] for c in range(WORD_LEN)]
        cnt = zero
        for c in range(WORD_LEN):
            cnt = cnt + jnp.where(toks[c] != 0, one, zero)
        inv = one / jnp.maximum(cnt, one)
        toks65 = [t * TSTRIDE for t in toks]
        for d in range(D):
            acc = plsc.load_gather(table_v, [toks65[0] + d])
            for c in range(1, WORD_LEN):
                acc = acc + plsc.load_gather(table_v, [toks65[c] + d])
            out_v[pl.ds(d * NPT + base, L)] = acc * inv
        return carry

    lax.fori_loop(0, NG, group_body, 0)

    pltpu.sync_copy(out_v, out_hbm.at[pl.ds(wid * NPT * D, NPT * D)])


@jax.jit
def _run(tokens_flat, idxe, table_flat):
    mesh = plsc.VectorSubcoreMesh(
        core_axis_name="c", subcore_axis_name="s",
        num_cores=NC, num_subcores=NS)
    f = pl.kernel(
        _sc_body,
        out_type=jax.ShapeDtypeStruct((N_NODES * D,), jnp.float32),
        mesh=mesh,
        compiler_params=pltpu.CompilerParams(needs_layout_passes=False),
        scratch_types=[
            pltpu.VMEM((N_CHUNKS, IDX_CHUNK), jnp.int32),
            pltpu.VMEM((N_ELEM,), jnp.int32),
            pltpu.VMEM((CHAR_VOCAB * TSTRIDE,), jnp.float32),
            pltpu.VMEM((D * NPT,), jnp.float32),
            pltpu.SemaphoreType.DMA,
        ],
    )
    out = f(tokens_flat, idxe, table_flat)
    return out.reshape(NW, D, NPT).transpose(0, 2, 1).reshape(N_NODES, D)


def kernel(local_char_embedding_tokens, lookup_ids, char_table, special_vectors):
    del special_vectors
    tokens_flat = local_char_embedding_tokens.astype(jnp.int32).reshape(-1)
    ids = lookup_ids.astype(jnp.int32).reshape(NW, 1, NPT)
    idxe = (ids * WORD_LEN
            + jnp.arange(WORD_LEN, dtype=jnp.int32).reshape(1, WORD_LEN, 1))
    idxe = idxe.reshape(NW, N_CHUNKS, IDX_CHUNK)
    table_flat = jnp.pad(char_table, ((0, 0), (0, TSTRIDE - D))).reshape(
        CHAR_VOCAB * TSTRIDE)
    return _run(tokens_flat, idxe, table_flat)


# speedup vs baseline: 31.9442x; 1.3916x over previous
import jax
import jax.numpy as jnp
from jax import lax
from jax.experimental import pallas as pl
from jax.experimental.pallas import tpu as pltpu
from jax.experimental.pallas import tpu_sc as plsc

WORD_LEN = 20
CHAR_VOCAB = 1000
D = 64
N_NODES = 16384
L = 16
NQ = D // L
NC, NS = 2, 16
NW = NC * NS
NPT = N_NODES // NW
NG = NPT // L
IDX_CHUNK = 128
N_ELEM = NPT * WORD_LEN
N_CHUNKS = N_ELEM // IDX_CHUNK
DMA_BATCH = 8
N_BATCHES = N_CHUNKS // DMA_BATCH
NODE_UNROLL = 4


def _sc_body(tokens_hbm, idxe_hbm, table_hbm, out_hbm,
             idxe_v, tok_v, table_v, out_v, sem):
    wid = lax.axis_index("s") * NC + lax.axis_index("c")

    pltpu.sync_copy(idxe_hbm.at[wid], idxe_v)
    pltpu.sync_copy(table_hbm, table_v)

    def dma_body(o, carry):
        copies = [
            pltpu.async_copy(
                tokens_hbm.at[idxe_v.at[o * DMA_BATCH + b]],
                tok_v.at[pl.ds((o * DMA_BATCH + b) * IDX_CHUNK, IDX_CHUNK)],
                sem)
            for b in range(DMA_BATCH)
        ]
        for cp in copies:
            cp.wait()
        return carry

    lax.fori_loop(0, N_BATCHES, dma_body, 0)

    zeros = jnp.zeros((L,), jnp.float32)
    for j in range(NQ):
        table_v[pl.ds(j * L, L)] = zeros

    one = jnp.ones((L,), jnp.float32)
    zero = jnp.zeros((L,), jnp.float32)

    def group_body(g, carry):
        base = g * L
        toks = [tok_v[pl.ds(c * NPT + base, L)] for c in range(WORD_LEN)]
        cnt = zero
        for c in range(WORD_LEN):
            cnt = cnt + jnp.where(toks[c] != 0, one, zero)
        inv = one / jnp.maximum(cnt, one)
        for u in range(L):
            accs = [zero] * NQ
            for c in range(WORD_LEN):
                t64 = toks[c][u] * D
                for q in range(NQ):
                    accs[q] = accs[q] + table_v[pl.ds(t64 + q * L, L)]
            inv_u = inv[u]
            for q in range(NQ):
                out_v[pl.ds((base + u) * D + q * L, L)] = accs[q] * inv_u
        return carry

    lax.fori_loop(0, NG, group_body, 0)

    pltpu.sync_copy(out_v, out_hbm.at[pl.ds(wid * NPT * D, NPT * D)])


@jax.jit
def _run(tokens_flat, idxe, table_flat):
    mesh = plsc.VectorSubcoreMesh(
        core_axis_name="c", subcore_axis_name="s",
        num_cores=NC, num_subcores=NS)
    f = pl.kernel(
        _sc_body,
        out_type=jax.ShapeDtypeStruct((N_NODES * D,), jnp.float32),
        mesh=mesh,
        compiler_params=pltpu.CompilerParams(needs_layout_passes=False),
        scratch_types=[
            pltpu.VMEM((N_CHUNKS, IDX_CHUNK), jnp.int32),
            pltpu.VMEM((N_ELEM,), jnp.int32),
            pltpu.VMEM((CHAR_VOCAB * D,), jnp.float32),
            pltpu.VMEM((NPT * D,), jnp.float32),
            pltpu.SemaphoreType.DMA,
        ],
    )
    return f(tokens_flat, idxe, table_flat).reshape(N_NODES, D)


def kernel(local_char_embedding_tokens, lookup_ids, char_table, special_vectors):
    del special_vectors
    tokens_flat = local_char_embedding_tokens.astype(jnp.int32).reshape(-1)
    ids = lookup_ids.astype(jnp.int32).reshape(NW, 1, NPT)
    idxe = (ids * WORD_LEN
            + jnp.arange(WORD_LEN, dtype=jnp.int32).reshape(1, WORD_LEN, 1))
    idxe = idxe.reshape(NW, N_CHUNKS, IDX_CHUNK)
    table_flat = char_table.reshape(CHAR_VOCAB * D)
    return _run(tokens_flat, idxe, table_flat)
